# Initial kernel scaffold; baseline (speedup 1.0000x reference)
import jax
import jax.numpy as jnp
from jax.experimental import pallas as pl


def kernel(obs, context, noise, W1, b1, W2, b2, W3, b3, Wm, bm):
    raise NotImplementedError("write your pallas kernel here")


# speedup vs baseline: 5.0651x; 5.0651x over previous
import jax
import jax.numpy as jnp
from jax.experimental import pallas as pl
from jax.experimental.pallas import tpu as pltpu

_TAU = 2.0
_V_TH = 1.0
_T_STEPS = 4
_HID = 1024
_ACT = 32
_OUT_MAX = 1.0


def _lif_step(v, pre):
    v = v + (pre - v) / _TAU
    spiked = v >= _V_TH
    s = jnp.where(spiked, 1.0, 0.0)
    v = jnp.where(spiked, 0.0, v)
    return v, s


def _spike_mlp_kernel(obs_ref, ctx_ref, w1o_ref, w1c_ref, b1_ref,
                      w2_ref, b2_ref, w3_ref, b3_ref, wm_ref, bm_ref,
                      noise_ref, am_ref, act_ref):
    pre1 = (jnp.dot(obs_ref[...], w1o_ref[...],
                    preferred_element_type=jnp.float32)
            + jnp.dot(ctx_ref[...], w1c_ref[...],
                      preferred_element_type=jnp.float32)
            + b1_ref[...])
    v1 = jnp.zeros_like(pre1)
    v2 = jnp.zeros_like(pre1)
    v3 = jnp.zeros_like(pre1)
    feat = jnp.zeros_like(pre1)
    for _ in range(_T_STEPS):
        v1, s1 = _lif_step(v1, pre1)
        pre2 = jnp.dot(s1.astype(jnp.bfloat16), w2_ref[...],
                       preferred_element_type=jnp.float32) + b2_ref[...]
        v2, s2 = _lif_step(v2, pre2)
        pre3 = jnp.dot(s2.astype(jnp.bfloat16), w3_ref[...],
                       preferred_element_type=jnp.float32) + b3_ref[...]
        v3, s3 = _lif_step(v3, pre3)
        feat = feat + s3
    feat = feat * (1.0 / _T_STEPS)
    logits = jnp.dot(feat.astype(jnp.bfloat16), wm_ref[...],
                     preferred_element_type=jnp.float32) + bm_ref[...]
    am = _OUT_MAX * jnp.tanh(logits)
    am_ref[...] = am
    act_ref[...] = am + jnp.clip(noise_ref[...], -0.1, 0.1)


def kernel(obs, context, noise, W1, b1, W2, b2, W3, b3, Wm, bm):
    B, obs_dim = obs.shape
    ctx_dim = context.shape[1]
    block_b = 512
    grid = (B // block_b,)

    w1o = W1[:, :obs_dim].T
    w1c = W1[:, obs_dim:].T
    w2 = W2.T.astype(jnp.bfloat16)
    w3 = W3.T.astype(jnp.bfloat16)
    wm = Wm.T.astype(jnp.bfloat16)
    b1r = b1.reshape(1, _HID)
    b2r = b2.reshape(1, _HID)
    b3r = b3.reshape(1, _HID)
    bmr = bm.reshape(1, _ACT)
    noiser = noise.reshape(1, _ACT)

    row_spec = lambda cols: pl.BlockSpec((block_b, cols), lambda i: (i, 0))
    full = lambda shape: pl.BlockSpec(shape, lambda i: (0, 0))

    out_shape = (
        jax.ShapeDtypeStruct((B, _ACT), jnp.float32),
        jax.ShapeDtypeStruct((B, _ACT), jnp.float32),
    )
    am, act = pl.pallas_call(
        _spike_mlp_kernel,
        grid=grid,
        in_specs=[
            row_spec(obs_dim),
            row_spec(ctx_dim),
            full((obs_dim, _HID)),
            full((ctx_dim, _HID)),
            full((1, _HID)),
            full((_HID, _HID)),
            full((1, _HID)),
            full((_HID, _HID)),
            full((1, _HID)),
            full((_HID, _ACT)),
            full((1, _ACT)),
            full((1, _ACT)),
        ],
        out_specs=(row_spec(_ACT), row_spec(_ACT)),
        out_shape=out_shape,
        compiler_params=pltpu.CompilerParams(
            dimension_semantics=("parallel",),
            vmem_limit_bytes=60 * 1024 * 1024,
        ),
        name="fused_lif_mlp",
    )(obs, context, w1o, w1c, b1r, w2, b2r, w3, b3r, wm, bmr, noiser)
    return (am, act)
